# baseline (speedup 1.0000x reference)
import functools

import jax
import jax.numpy as jnp
from jax import lax
from jax.experimental import pallas as pl
from jax.experimental.pallas import tpu as pltpu
from jax.experimental.pallas import tpu_sc as plsc

H = 2
C = 30
HC = H * C
NEG_SLOPE = 0.2

NCORE = 2
NSUB = 16
NPASS = 3
B_N = 16768
ACC_ROWS = NCORE * NPASS * B_N
SHARE = B_N // NSUB
EBLK = 2000
PROWS = 16
DRAIN = 256


def _leaky(x):
    return jnp.where(x >= 0, x, NEG_SLOPE * x)


def _stats_body(h_ref, o_ref):
    h = h_ref[...]
    n = h.shape[0]
    s = jnp.sum(h, axis=0, keepdims=True)
    sq = jnp.sum(h * h, axis=0, keepdims=True)
    mean = s / n
    var = sq / n - mean * mean
    o_ref[...] = jnp.concatenate([mean, var], axis=0)


def _stats(h):
    return pl.pallas_call(
        _stats_body,
        out_shape=jax.ShapeDtypeStruct((2, h.shape[1]), jnp.float32),
    )(h)


def _xtab_body(h_ref, st_ref, m_ref, gb_ref, o_ref, o2_ref):
    h = h_ref[...]
    mean = st_ref[0:1, :]
    var = st_ref[1:2, :]
    gamma = gb_ref[0:1, :]
    beta = gb_ref[1:2, :]
    hn = (h - mean) * jax.lax.rsqrt(var + 1e-5) * gamma + beta
    prod = jnp.dot(hn, m_ref[...], preferred_element_type=jnp.float32)
    o_ref[...] = prod
    o2_ref[...] = prod[:, 62:64]


def _xtab(h, stats, m, gb, rb, n_pad):
    n = h.shape[0]
    grid = (n + rb - 1) // rb
    return pl.pallas_call(
        _xtab_body,
        grid=(grid,),
        in_specs=[
            pl.BlockSpec((rb, h.shape[1]), lambda i: (i, 0)),
            pl.BlockSpec((2, h.shape[1]), lambda i: (0, 0)),
            pl.BlockSpec((h.shape[1], 64), lambda i: (0, 0)),
            pl.BlockSpec((2, h.shape[1]), lambda i: (0, 0)),
        ],
        out_specs=[
            pl.BlockSpec((rb, 64), lambda i: (i, 0)),
            pl.BlockSpec((rb, 2), lambda i: (i, 0)),
        ],
        out_shape=[
            jax.ShapeDtypeStruct((n, 64), jnp.float32),
            jax.ShapeDtypeStruct((n_pad, 2), jnp.float32),
        ],
    )(h, stats, m, gb)


def _edge_body(src_hbm, dst_hbm, ew_hbm, xtab_hbm, adst_hbm, c2_hbm, acc_hbm,
               bs, bd, bw, psrc, pdl, pew, xrows, arows, ppe0, ppe1, zrows,
               c2_v, acc_sp, adst_sp, sem1, sem2):
    core = lax.axis_index("c")
    tid = lax.axis_index("s")
    e_total = src_hbm.shape[0]
    per_tile = e_total // NSUB
    nblk = per_tile // EBLK
    vpb = EBLK // 16

    pltpu.sync_copy(c2_hbm, c2_v)
    c2v = c2_v[pl.ds(0, 16)]
    c0b = jnp.full((16,), c2v[0], jnp.float32)
    c1b = jnp.full((16,), c2v[1], jnp.float32)
    iota16 = lax.iota(jnp.int32, 16)
    zero16 = jnp.zeros((16,), jnp.float32)
    one16 = jnp.ones((16,), jnp.float32)
    zero16i = jnp.zeros((16,), jnp.int32)
    col60 = jnp.full((16,), 60, jnp.int32)
    col61 = jnp.full((16,), 61, jnp.int32)
    acol0 = jnp.zeros((16,), jnp.int32)
    acol1 = jnp.full((16,), 1, jnp.int32)
    bn16 = jnp.full((16,), B_N, jnp.int32)
    slope16 = jnp.full((16,), NEG_SLOPE, jnp.float32)
    pm16 = jnp.full((16,), PROWS - 1, jnp.int32)
    cm16 = jnp.full((16,), DRAIN - 1, jnp.int32)
    sh16 = jnp.full((16,), 8, jnp.int32)

    def zr_body(r, carry):
        for k in range(4):
            zrows[r, pl.ds(16 * k, 16)] = zero16
        return carry

    lax.fori_loop(0, 256, zr_body, 0)

    def drain(r):
        cp1 = pltpu.async_copy(xtab_hbm.at[psrc.at[r]], xrows, sem1)
        cp2 = pltpu.async_copy(adst_sp.at[pdl.at[r]], arows, sem2)
        cp1.wait()
        cp2.wait()
        def pe_body(j, carry):
            ei = jnp.full((16,), j * 16, jnp.int32) + iota16
            a0 = plsc.load_gather(xrows, [ei, col60])
            a1 = plsc.load_gather(xrows, [ei, col61])
            b0 = plsc.load_gather(arows, [ei, acol0])
            b1 = plsc.load_gather(arows, [ei, acol1])
            w = pew[r, pl.ds(j * 16, 16)]
            l0 = a0 + b0 + c0b * w
            l0 = jnp.where(l0 >= zero16, l0, slope16 * l0)
            l1 = a1 + b1 + c1b * w
            l1 = jnp.where(l1 >= zero16, l1, slope16 * l1)
            ppe0[pl.ds(j * 16, 16)] = jnp.exp(l0)
            ppe1[pl.ds(j * 16, 16)] = jnp.exp(l1)
            return carry

        lax.fori_loop(0, DRAIN // 16, pe_body, 0)

        def mul_body(j, carry):
            ev = jnp.full((16,), j * 16, jnp.int32) + iota16
            p0v = ppe0[pl.ds(j * 16, 16)]
            p1v = ppe1[pl.ds(j * 16, 16)]
            wv = pew[r, pl.ds(j * 16, 16)]
            for c in range(30):
                colv = jnp.full((16,), c, jnp.int32)
                v = plsc.load_gather(xrows, [ev, colv])
                plsc.store_scatter(xrows, [ev, colv], v * p0v)
            for c in range(30, 60):
                colv = jnp.full((16,), c, jnp.int32)
                v = plsc.load_gather(xrows, [ev, colv])
                plsc.store_scatter(xrows, [ev, colv], v * p1v)
            plsc.store_scatter(xrows, [ev, col60], p0v)
            plsc.store_scatter(xrows, [ev, col61], p1v)
            plsc.store_scatter(xrows, [ev, jnp.full((16,), 62, jnp.int32)],
                               one16)
            plsc.store_scatter(xrows, [ev, jnp.full((16,), 63, jnp.int32)],
                               wv)
            return carry

        lax.fori_loop(0, DRAIN // 16, mul_body, 0)
        pltpu.sync_copy(xrows, acc_sp.at[pdl.at[r]], add=True)

    for p in range(NPASS):
        base = (NPASS * core + p) * B_N
        for q in range(SHARE // 256):
            pltpu.sync_copy(zrows, acc_sp.at[pl.ds(tid * SHARE + q * 256, 256)])
        if SHARE % 256:
            pltpu.sync_copy(
                zrows.at[pl.ds(0, SHARE % 256)],
                acc_sp.at[pl.ds(tid * SHARE + SHARE - SHARE % 256,
                                SHARE % 256)])
        pltpu.sync_copy(adst_hbm.at[pl.ds(base + tid * SHARE, SHARE)],
                        adst_sp.at[pl.ds(tid * SHARE, SHARE)])
        plsc.subcore_barrier()

        def blk_body(blk, carry):
            pend, done = carry
            off = tid * per_tile + blk * EBLK
            pltpu.sync_copy(src_hbm.at[pl.ds(off, EBLK)], bs)
            pltpu.sync_copy(dst_hbm.at[pl.ds(off, EBLK)], bd)
            pltpu.sync_copy(ew_hbm.at[pl.ds(off, EBLK)], bw)

            baseb = jnp.full((16,), base, jnp.int32)

            def vreg_body(i, pend):
                s = bs[pl.ds(i * 16, 16)]
                d = bd[pl.ds(i * 16, 16)]
                w = bw[pl.ds(i * 16, 16)]
                dl = d - baseb
                m = (dl >= zero16i) & (dl < bn16)
                mi = m.astype(jnp.int32)
                cum = plsc.cumsum(mi)
                pos = jnp.full((16,), pend, jnp.int32) + cum - mi
                row = (pos >> sh16) & pm16
                col = pos & cm16
                plsc.store_scatter(psrc, [row, col], s, mask=m)
                plsc.store_scatter(pdl, [row, col], dl, mask=m)
                plsc.store_scatter(pew, [row, col], w, mask=m)
                return pend + cum[15]

            pend = lax.fori_loop(0, vpb, vreg_body, pend)

            def dcond(c):
                return c[0] - c[1] >= DRAIN

            def dbody(c):
                pend, done = c
                drain((done >> 8) & (PROWS - 1))
                return (pend, done + DRAIN)

            pend, done = lax.while_loop(dcond, dbody, (pend, done))
            return (pend, done)

        pend, done = lax.fori_loop(0, nblk, blk_body,
                                   (jnp.int32(0), jnp.int32(0)))

        @pl.when(pend > done)
        def _():
            for j in range(DRAIN // 16):
                pos = jnp.full((16,), pend + j * 16, jnp.int32) + iota16
                row = (pos >> sh16) & pm16
                col = pos & cm16
                plsc.store_scatter(psrc, [row, col], zero16i)
                plsc.store_scatter(pdl, [row, col], bn16)
                plsc.store_scatter(pew, [row, col], zero16)
            drain((done >> 8) & (PROWS - 1))

        plsc.subcore_barrier()
        for q in range(SHARE // 256):
            pltpu.sync_copy(acc_sp.at[pl.ds(tid * SHARE + q * 256, 256)],
                            acc_hbm.at[pl.ds(base + tid * SHARE + q * 256, 256)])
        if SHARE % 256:
            pltpu.sync_copy(
                acc_sp.at[pl.ds(tid * SHARE + SHARE - SHARE % 256,
                                SHARE % 256)],
                acc_hbm.at[pl.ds(base + tid * SHARE + SHARE - SHARE % 256,
                                 SHARE % 256)])
        if p + 1 < NPASS:
            plsc.subcore_barrier()


def _edge_phase(src, dst, ew, xtab, adst_pad, c2pad):
    mesh = plsc.VectorSubcoreMesh(core_axis_name="c", subcore_axis_name="s")
    call = pl.kernel(
        _edge_body,
        out_type=jax.ShapeDtypeStruct((ACC_ROWS, 64), jnp.float32),
        mesh=mesh,
        compiler_params=pltpu.CompilerParams(
            needs_layout_passes=False, use_tc_tiling_on_sc=False),
        scratch_types=[
            pltpu.VMEM((EBLK,), jnp.int32),
            pltpu.VMEM((EBLK,), jnp.int32),
            pltpu.VMEM((EBLK,), jnp.float32),
            pltpu.VMEM((PROWS, DRAIN), jnp.int32),
            pltpu.VMEM((PROWS, DRAIN), jnp.int32),
            pltpu.VMEM((PROWS, DRAIN), jnp.float32),
            pltpu.VMEM((DRAIN, 64), jnp.float32),
            pltpu.VMEM((DRAIN, 2), jnp.float32),
            pltpu.VMEM((DRAIN,), jnp.float32),
            pltpu.VMEM((DRAIN,), jnp.float32),
            pltpu.VMEM((256, 64), jnp.float32),
            pltpu.VMEM((16,), jnp.float32),
            pltpu.VMEM_SHARED((B_N + 8, 64), jnp.float32),
            pltpu.VMEM_SHARED((B_N + 8, 2), jnp.float32),
            pltpu.SemaphoreType.DMA,
            pltpu.SemaphoreType.DMA,
        ],
    )
    return call(src, dst, ew, xtab, adst_pad, c2pad)


def _head_body(acc_ref, xt_ref, c2_ref, wb_ref, a_ref, b_ref):
    acc = acc_ref[...]
    xt = xt_ref[...]
    rb = acc.shape[0]
    c2 = c2_ref[0, :]

    counts = acc[:, 62]
    wsum = acc[:, 63]
    la = wsum / jnp.maximum(counts, 1.0)
    asrc = xt[:, 60:62]
    adst = xt[:, 62:64]
    pes = jnp.exp(_leaky(asrc + adst + la[:, None] * c2[None, 0:2]))
    den = acc[:, 60:62] + pes
    pe_cols = jnp.concatenate(
        [jnp.broadcast_to(pes[:, 0:1], (rb, C)),
         jnp.broadcast_to(pes[:, 1:2], (rb, C))], axis=1)
    den_cols = jnp.concatenate(
        [jnp.broadcast_to(den[:, 0:1], (rb, C)),
         jnp.broadcast_to(den[:, 1:2], (rb, C))], axis=1)
    out = (acc[:, 0:HC] + pe_cols * xt[:, 0:HC]) / den_cols

    wb = wb_ref[...]
    hh = jnp.maximum(out + wb[0:1, 0:HC], 0.0)
    hh = jnp.maximum(
        jnp.dot(hh, wb[1:61, 0:10], preferred_element_type=jnp.float32)
        + wb[61:62, 0:10], 0.0)
    hh = jnp.maximum(
        jnp.dot(hh, wb[62:72, 0:10], preferred_element_type=jnp.float32)
        + wb[72:73, 0:10], 0.0)
    hh = (jnp.dot(hh, wb[73:83, 0:10], preferred_element_type=jnp.float32)
          + wb[83:84, 0:10])
    ab = (jnp.dot(hh, wb[84:94, 0:2], preferred_element_type=jnp.float32)
          + wb[94:95, 0:2])
    ab = jnp.where(ab > 0, ab, jnp.exp(jnp.minimum(ab, 0.0)) - 1.0) + 1.0
    a_ref[...] = ab[:, 0:1]
    b_ref[...] = ab[:, 1:2]


def _head(acc, xtab, c2, wb, rb):
    n = xtab.shape[0]
    grid = (n + rb - 1) // rb
    return pl.pallas_call(
        _head_body,
        grid=(grid,),
        in_specs=[
            pl.BlockSpec((rb, 64), lambda i: (i, 0)),
            pl.BlockSpec((rb, 64), lambda i: (i, 0)),
            pl.BlockSpec((1, 2), lambda i: (0, 0)),
            pl.BlockSpec((95, 64), lambda i: (0, 0)),
        ],
        out_specs=[
            pl.BlockSpec((rb, 1), lambda i: (i, 0)),
            pl.BlockSpec((rb, 1), lambda i: (i, 0)),
        ],
        out_shape=[
            jax.ShapeDtypeStruct((n, 1), jnp.float32),
            jax.ShapeDtypeStruct((n, 1), jnp.float32),
        ],
    )(acc, xtab, c2, wb)


def _pack_head_weights(params):
    rows = []

    def pad(row2d):
        r, c = row2d.shape
        return jnp.pad(row2d, ((0, 0), (0, 64 - c)))

    rows.append(pad(params['conv_bias'][None, :]))
    rows.append(pad(params['fc1_W'].T))
    rows.append(pad(params['fc1_b'][None, :]))
    rows.append(pad(params['fc2_W'].T))
    rows.append(pad(params['fc2_b'][None, :]))
    rows.append(pad(params['fc3_W'].T))
    rows.append(pad(params['fc3_b'][None, :]))
    fc45 = jnp.concatenate([params['fc4_W'], params['fc5_W']], axis=0)
    rows.append(pad(fc45.T))
    fc45b = jnp.concatenate([params['fc4_b'], params['fc5_b']])[None, :]
    rows.append(pad(fc45b))
    return jnp.concatenate(rows, axis=0)


def kernel(h, edge_index, edge_weight, params):
    n, in_dim = h.shape

    wt = params['lin_W'].T
    att_s = params['att_src'][0]
    att_d = params['att_dst'][0]
    sel_s = jnp.zeros((HC, H), jnp.float32)
    sel_d = jnp.zeros((HC, H), jnp.float32)
    for hh in range(H):
        sel_s = sel_s.at[hh * C:(hh + 1) * C, hh].set(att_s[hh])
        sel_d = sel_d.at[hh * C:(hh + 1) * C, hh].set(att_d[hh])
    m = jnp.concatenate([wt, wt @ sel_s, wt @ sel_d], axis=1)
    gb = jnp.stack([params['bn_gamma'], params['bn_beta']])
    le = params['lin_edge_W'][:, 0].reshape(H, C)
    c2 = jnp.sum(le * params['att_edge'][0], axis=-1)[None, :]

    stats = _stats(h)
    xtab, adst_pad = _xtab(h, stats, m, gb, rb=8192, n_pad=ACC_ROWS)

    src, dst = edge_index[0], edge_index[1]
    ew = edge_weight[:, 0]
    c2pad = jnp.pad(c2[0], (0, 14))
    acc_full = _edge_phase(src, dst, ew, xtab, adst_pad, c2pad)

    wb = _pack_head_weights(params)
    a_out, b_out = _head(acc_full, xtab, c2, wb, rb=8192)
    return a_out, b_out


# speedup vs baseline: 2.5196x; 2.5196x over previous
import functools

import jax
import jax.numpy as jnp
from jax import lax
from jax.experimental import pallas as pl
from jax.experimental.pallas import tpu as pltpu
from jax.experimental.pallas import tpu_sc as plsc

H = 2
C = 30
HC = H * C
NEG_SLOPE = 0.2

NCORE = 2
NSUB = 16
NPASS = 3
B_N = 16768
ACC_ROWS = NCORE * NPASS * B_N
SHARE = B_N // NSUB
EBLK = 2000
PROWS = 16
DRAIN = 256


def _leaky(x):
    return jnp.where(x >= 0, x, NEG_SLOPE * x)


def _stats_body(h_ref, o_ref):
    h = h_ref[...]
    n = h.shape[0]
    s = jnp.sum(h, axis=0, keepdims=True)
    sq = jnp.sum(h * h, axis=0, keepdims=True)
    mean = s / n
    var = sq / n - mean * mean
    o_ref[...] = jnp.concatenate([mean, var], axis=0)


def _stats(h):
    return pl.pallas_call(
        _stats_body,
        out_shape=jax.ShapeDtypeStruct((2, h.shape[1]), jnp.float32),
    )(h)


def _xtab_body(h_ref, st_ref, m_ref, gb_ref, o_ref, o2_ref):
    h = h_ref[...]
    mean = st_ref[0:1, :]
    var = st_ref[1:2, :]
    gamma = gb_ref[0:1, :]
    beta = gb_ref[1:2, :]
    hn = (h - mean) * jax.lax.rsqrt(var + 1e-5) * gamma + beta
    prod = jnp.dot(hn, m_ref[...], preferred_element_type=jnp.float32)
    o_ref[...] = prod
    o2_ref[...] = prod[:, 62:64]


def _xtab(h, stats, m, gb, rb, n_pad):
    n = h.shape[0]
    grid = (n + rb - 1) // rb
    return pl.pallas_call(
        _xtab_body,
        grid=(grid,),
        in_specs=[
            pl.BlockSpec((rb, h.shape[1]), lambda i: (i, 0)),
            pl.BlockSpec((2, h.shape[1]), lambda i: (0, 0)),
            pl.BlockSpec((h.shape[1], 64), lambda i: (0, 0)),
            pl.BlockSpec((2, h.shape[1]), lambda i: (0, 0)),
        ],
        out_specs=[
            pl.BlockSpec((rb, 64), lambda i: (i, 0)),
            pl.BlockSpec((rb, 2), lambda i: (i, 0)),
        ],
        out_shape=[
            jax.ShapeDtypeStruct((n, 64), jnp.float32),
            jax.ShapeDtypeStruct((n_pad, 2), jnp.float32),
        ],
    )(h, stats, m, gb)


def _edge_body(src_hbm, dst_hbm, ew_hbm, xtab_hbm, adst_hbm, c2_hbm, acc_hbm,
               bs, bd, bw, psrc, pdl, pew, xrows, arows, ppe0, ppe1, zrows,
               c2_v, acc_sp, adst_sp, sem1, sem2):
    core = lax.axis_index("c")
    tid = lax.axis_index("s")
    e_total = src_hbm.shape[0]
    per_tile = e_total // NSUB
    nblk = per_tile // EBLK
    vpb = EBLK // 16

    pltpu.sync_copy(c2_hbm, c2_v)
    c2v = c2_v[pl.ds(0, 16)]
    c0b = jnp.full((16,), c2v[0], jnp.float32)
    c1b = jnp.full((16,), c2v[1], jnp.float32)
    iota16 = lax.iota(jnp.int32, 16)
    zero16 = jnp.zeros((16,), jnp.float32)
    one16 = jnp.ones((16,), jnp.float32)
    zero16i = jnp.zeros((16,), jnp.int32)
    col60 = jnp.full((16,), 60, jnp.int32)
    col61 = jnp.full((16,), 61, jnp.int32)
    acol0 = jnp.zeros((16,), jnp.int32)
    acol1 = jnp.full((16,), 1, jnp.int32)
    bn16 = jnp.full((16,), B_N, jnp.int32)
    slope16 = jnp.full((16,), NEG_SLOPE, jnp.float32)
    pm16 = jnp.full((16,), PROWS - 1, jnp.int32)
    cm16 = jnp.full((16,), DRAIN - 1, jnp.int32)
    sh16 = jnp.full((16,), 8, jnp.int32)

    def zr_body(r, carry):
        for k in range(4):
            zrows[r, pl.ds(16 * k, 16)] = zero16
        return carry

    lax.fori_loop(0, 256, zr_body, 0)

    def drain(r):
        cp1 = pltpu.async_copy(xtab_hbm.at[psrc.at[r]], xrows, sem1)
        cp2 = pltpu.async_copy(adst_sp.at[pdl.at[r]], arows, sem2)
        cp1.wait()
        cp2.wait()
        def pe_body(j, carry):
            ei = jnp.full((16,), j * 16, jnp.int32) + iota16
            a0 = plsc.load_gather(xrows, [ei, col60])
            a1 = plsc.load_gather(xrows, [ei, col61])
            b0 = plsc.load_gather(arows, [ei, acol0])
            b1 = plsc.load_gather(arows, [ei, acol1])
            w = pew[r, pl.ds(j * 16, 16)]
            l0 = a0 + b0 + c0b * w
            l0 = jnp.where(l0 >= zero16, l0, slope16 * l0)
            l1 = a1 + b1 + c1b * w
            l1 = jnp.where(l1 >= zero16, l1, slope16 * l1)
            ppe0[pl.ds(j * 16, 16)] = jnp.exp(l0)
            ppe1[pl.ds(j * 16, 16)] = jnp.exp(l1)
            return carry

        lax.fori_loop(0, DRAIN // 16, pe_body, 0)

        def mul_body(j, carry):
            ev = jnp.full((16,), j * 16, jnp.int32) + iota16
            p0v = ppe0[pl.ds(j * 16, 16)]
            p1v = ppe1[pl.ds(j * 16, 16)]
            wv = pew[r, pl.ds(j * 16, 16)]
            i14 = jnp.full((16,), 14, jnp.int32)
            for e2 in range(16):
                e = j * 16 + e2
                p0 = jnp.full((16,), p0v[e2], jnp.float32)
                p1 = jnp.full((16,), p1v[e2], jnp.float32)
                c1 = jnp.where(iota16 < i14, p0, p1)
                v0 = xrows[e, pl.ds(0, 16)]
                xrows[e, pl.ds(0, 16)] = v0 * p0
                v1 = xrows[e, pl.ds(16, 16)]
                xrows[e, pl.ds(16, 16)] = v1 * c1
                v2 = xrows[e, pl.ds(32, 16)]
                xrows[e, pl.ds(32, 16)] = v2 * p1
                v3 = xrows[e, pl.ds(48, 16)]
                xrows[e, pl.ds(48, 16)] = v3 * p1
            plsc.store_scatter(xrows, [ev, col60], p0v)
            plsc.store_scatter(xrows, [ev, col61], p1v)
            plsc.store_scatter(xrows, [ev, jnp.full((16,), 62, jnp.int32)],
                               one16)
            plsc.store_scatter(xrows, [ev, jnp.full((16,), 63, jnp.int32)],
                               wv)
            return carry

        lax.fori_loop(0, DRAIN // 16, mul_body, 0)
        pltpu.sync_copy(xrows, acc_sp.at[pdl.at[r]], add=True)

    for p in range(NPASS):
        base = (NPASS * core + p) * B_N
        for q in range(SHARE // 256):
            pltpu.sync_copy(zrows, acc_sp.at[pl.ds(tid * SHARE + q * 256, 256)])
        if SHARE % 256:
            pltpu.sync_copy(
                zrows.at[pl.ds(0, SHARE % 256)],
                acc_sp.at[pl.ds(tid * SHARE + SHARE - SHARE % 256,
                                SHARE % 256)])
        pltpu.sync_copy(adst_hbm.at[pl.ds(base + tid * SHARE, SHARE)],
                        adst_sp.at[pl.ds(tid * SHARE, SHARE)])
        plsc.subcore_barrier()

        def blk_body(blk, carry):
            pend, done = carry
            off = tid * per_tile + blk * EBLK
            pltpu.sync_copy(src_hbm.at[pl.ds(off, EBLK)], bs)
            pltpu.sync_copy(dst_hbm.at[pl.ds(off, EBLK)], bd)
            pltpu.sync_copy(ew_hbm.at[pl.ds(off, EBLK)], bw)

            baseb = jnp.full((16,), base, jnp.int32)

            def vreg_body(i, pend):
                s = bs[pl.ds(i * 16, 16)]
                d = bd[pl.ds(i * 16, 16)]
                w = bw[pl.ds(i * 16, 16)]
                dl = d - baseb
                m = (dl >= zero16i) & (dl < bn16)
                mi = m.astype(jnp.int32)
                cum = plsc.cumsum(mi)
                pos = jnp.full((16,), pend, jnp.int32) + cum - mi
                row = (pos >> sh16) & pm16
                col = pos & cm16
                plsc.store_scatter(psrc, [row, col], s, mask=m)
                plsc.store_scatter(pdl, [row, col], dl, mask=m)
                plsc.store_scatter(pew, [row, col], w, mask=m)
                return pend + cum[15]

            pend = lax.fori_loop(0, vpb, vreg_body, pend)

            def dcond(c):
                return c[0] - c[1] >= DRAIN

            def dbody(c):
                pend, done = c
                drain((done >> 8) & (PROWS - 1))
                return (pend, done + DRAIN)

            pend, done = lax.while_loop(dcond, dbody, (pend, done))
            return (pend, done)

        pend, done = lax.fori_loop(0, nblk, blk_body,
                                   (jnp.int32(0), jnp.int32(0)))

        @pl.when(pend > done)
        def _():
            for j in range(DRAIN // 16):
                pos = jnp.full((16,), pend + j * 16, jnp.int32) + iota16
                row = (pos >> sh16) & pm16
                col = pos & cm16
                plsc.store_scatter(psrc, [row, col], zero16i)
                plsc.store_scatter(pdl, [row, col], bn16)
                plsc.store_scatter(pew, [row, col], zero16)
            drain((done >> 8) & (PROWS - 1))

        plsc.subcore_barrier()
        for q in range(SHARE // 256):
            pltpu.sync_copy(acc_sp.at[pl.ds(tid * SHARE + q * 256, 256)],
                            acc_hbm.at[pl.ds(base + tid * SHARE + q * 256, 256)])
        if SHARE % 256:
            pltpu.sync_copy(
                acc_sp.at[pl.ds(tid * SHARE + SHARE - SHARE % 256,
                                SHARE % 256)],
                acc_hbm.at[pl.ds(base + tid * SHARE + SHARE - SHARE % 256,
                                 SHARE % 256)])
        if p + 1 < NPASS:
            plsc.subcore_barrier()


def _edge_phase(src, dst, ew, xtab, adst_pad, c2pad):
    mesh = plsc.VectorSubcoreMesh(core_axis_name="c", subcore_axis_name="s")
    call = pl.kernel(
        _edge_body,
        out_type=jax.ShapeDtypeStruct((ACC_ROWS, 64), jnp.float32),
        mesh=mesh,
        compiler_params=pltpu.CompilerParams(
            needs_layout_passes=False, use_tc_tiling_on_sc=False),
        scratch_types=[
            pltpu.VMEM((EBLK,), jnp.int32),
            pltpu.VMEM((EBLK,), jnp.int32),
            pltpu.VMEM((EBLK,), jnp.float32),
            pltpu.VMEM((PROWS, DRAIN), jnp.int32),
            pltpu.VMEM((PROWS, DRAIN), jnp.int32),
            pltpu.VMEM((PROWS, DRAIN), jnp.float32),
            pltpu.VMEM((DRAIN, 64), jnp.float32),
            pltpu.VMEM((DRAIN, 2), jnp.float32),
            pltpu.VMEM((DRAIN,), jnp.float32),
            pltpu.VMEM((DRAIN,), jnp.float32),
            pltpu.VMEM((256, 64), jnp.float32),
            pltpu.VMEM((16,), jnp.float32),
            pltpu.VMEM_SHARED((B_N + 8, 64), jnp.float32),
            pltpu.VMEM_SHARED((B_N + 8, 2), jnp.float32),
            pltpu.SemaphoreType.DMA,
            pltpu.SemaphoreType.DMA,
        ],
    )
    return call(src, dst, ew, xtab, adst_pad, c2pad)


def _head_body(acc_ref, xt_ref, c2_ref, wb_ref, a_ref, b_ref):
    acc = acc_ref[...]
    xt = xt_ref[...]
    rb = acc.shape[0]
    c2 = c2_ref[0, :]

    counts = acc[:, 62]
    wsum = acc[:, 63]
    la = wsum / jnp.maximum(counts, 1.0)
    asrc = xt[:, 60:62]
    adst = xt[:, 62:64]
    pes = jnp.exp(_leaky(asrc + adst + la[:, None] * c2[None, 0:2]))
    den = acc[:, 60:62] + pes
    pe_cols = jnp.concatenate(
        [jnp.broadcast_to(pes[:, 0:1], (rb, C)),
         jnp.broadcast_to(pes[:, 1:2], (rb, C))], axis=1)
    den_cols = jnp.concatenate(
        [jnp.broadcast_to(den[:, 0:1], (rb, C)),
         jnp.broadcast_to(den[:, 1:2], (rb, C))], axis=1)
    out = (acc[:, 0:HC] + pe_cols * xt[:, 0:HC]) / den_cols

    wb = wb_ref[...]
    hh = jnp.maximum(out + wb[0:1, 0:HC], 0.0)
    hh = jnp.maximum(
        jnp.dot(hh, wb[1:61, 0:10], preferred_element_type=jnp.float32)
        + wb[61:62, 0:10], 0.0)
    hh = jnp.maximum(
        jnp.dot(hh, wb[62:72, 0:10], preferred_element_type=jnp.float32)
        + wb[72:73, 0:10], 0.0)
    hh = (jnp.dot(hh, wb[73:83, 0:10], preferred_element_type=jnp.float32)
          + wb[83:84, 0:10])
    ab = (jnp.dot(hh, wb[84:94, 0:2], preferred_element_type=jnp.float32)
          + wb[94:95, 0:2])
    ab = jnp.where(ab > 0, ab, jnp.exp(jnp.minimum(ab, 0.0)) - 1.0) + 1.0
    a_ref[...] = ab[:, 0:1]
    b_ref[...] = ab[:, 1:2]


def _head(acc, xtab, c2, wb, rb):
    n = xtab.shape[0]
    grid = (n + rb - 1) // rb
    return pl.pallas_call(
        _head_body,
        grid=(grid,),
        in_specs=[
            pl.BlockSpec((rb, 64), lambda i: (i, 0)),
            pl.BlockSpec((rb, 64), lambda i: (i, 0)),
            pl.BlockSpec((1, 2), lambda i: (0, 0)),
            pl.BlockSpec((95, 64), lambda i: (0, 0)),
        ],
        out_specs=[
            pl.BlockSpec((rb, 1), lambda i: (i, 0)),
            pl.BlockSpec((rb, 1), lambda i: (i, 0)),
        ],
        out_shape=[
            jax.ShapeDtypeStruct((n, 1), jnp.float32),
            jax.ShapeDtypeStruct((n, 1), jnp.float32),
        ],
    )(acc, xtab, c2, wb)


def _pack_head_weights(params):
    rows = []

    def pad(row2d):
        r, c = row2d.shape
        return jnp.pad(row2d, ((0, 0), (0, 64 - c)))

    rows.append(pad(params['conv_bias'][None, :]))
    rows.append(pad(params['fc1_W'].T))
    rows.append(pad(params['fc1_b'][None, :]))
    rows.append(pad(params['fc2_W'].T))
    rows.append(pad(params['fc2_b'][None, :]))
    rows.append(pad(params['fc3_W'].T))
    rows.append(pad(params['fc3_b'][None, :]))
    fc45 = jnp.concatenate([params['fc4_W'], params['fc5_W']], axis=0)
    rows.append(pad(fc45.T))
    fc45b = jnp.concatenate([params['fc4_b'], params['fc5_b']])[None, :]
    rows.append(pad(fc45b))
    return jnp.concatenate(rows, axis=0)


def kernel(h, edge_index, edge_weight, params):
    n, in_dim = h.shape

    wt = params['lin_W'].T
    att_s = params['att_src'][0]
    att_d = params['att_dst'][0]
    sel_s = jnp.zeros((HC, H), jnp.float32)
    sel_d = jnp.zeros((HC, H), jnp.float32)
    for hh in range(H):
        sel_s = sel_s.at[hh * C:(hh + 1) * C, hh].set(att_s[hh])
        sel_d = sel_d.at[hh * C:(hh + 1) * C, hh].set(att_d[hh])
    m = jnp.concatenate([wt, wt @ sel_s, wt @ sel_d], axis=1)
    gb = jnp.stack([params['bn_gamma'], params['bn_beta']])
    le = params['lin_edge_W'][:, 0].reshape(H, C)
    c2 = jnp.sum(le * params['att_edge'][0], axis=-1)[None, :]

    stats = _stats(h)
    xtab, adst_pad = _xtab(h, stats, m, gb, rb=8192, n_pad=ACC_ROWS)

    src, dst = edge_index[0], edge_index[1]
    ew = edge_weight[:, 0]
    c2pad = jnp.pad(c2[0], (0, 14))
    acc_full = _edge_phase(src, dst, ew, xtab, adst_pad, c2pad)

    wb = _pack_head_weights(params)
    a_out, b_out = _head(acc_full, xtab, c2, wb, rb=8192)
    return a_out, b_out


# speedup vs baseline: 2.7029x; 1.0727x over previous
import functools

import jax
import jax.numpy as jnp
from jax import lax
from jax.experimental import pallas as pl
from jax.experimental.pallas import tpu as pltpu
from jax.experimental.pallas import tpu_sc as plsc

H = 2
C = 30
HC = H * C
NEG_SLOPE = 0.2

NCORE = 2
NSUB = 16
NPASS = 3
B_N = 16768
ACC_ROWS = NCORE * NPASS * B_N
SHARE = B_N // NSUB
EBLK = 4000
PROWS = 16
DRAIN = 256


def _leaky(x):
    return jnp.where(x >= 0, x, NEG_SLOPE * x)


def _stats_body(h_ref, o_ref):
    h = h_ref[...]
    n = h.shape[0]
    s = jnp.sum(h, axis=0, keepdims=True)
    sq = jnp.sum(h * h, axis=0, keepdims=True)
    mean = s / n
    var = sq / n - mean * mean
    o_ref[...] = jnp.concatenate([mean, var], axis=0)


def _stats(h):
    return pl.pallas_call(
        _stats_body,
        out_shape=jax.ShapeDtypeStruct((2, h.shape[1]), jnp.float32),
    )(h)


def _xtab_body(h_ref, st_ref, m_ref, gb_ref, o_ref, o2_ref, o3_ref):
    h = h_ref[...]
    mean = st_ref[0:1, :]
    var = st_ref[1:2, :]
    gamma = gb_ref[0:1, :]
    beta = gb_ref[1:2, :]
    hn = (h - mean) * jax.lax.rsqrt(var + 1e-5) * gamma + beta
    prod = jnp.dot(hn, m_ref[...], preferred_element_type=jnp.float32)
    o_ref[...] = prod
    o2_ref[...] = prod[:, 62:64]
    o3_ref[...] = prod[:, 60:62]


def _xtab(h, stats, m, gb, rb, n_pad):
    n = h.shape[0]
    grid = (n + rb - 1) // rb
    return pl.pallas_call(
        _xtab_body,
        grid=(grid,),
        in_specs=[
            pl.BlockSpec((rb, h.shape[1]), lambda i: (i, 0)),
            pl.BlockSpec((2, h.shape[1]), lambda i: (0, 0)),
            pl.BlockSpec((h.shape[1], 64), lambda i: (0, 0)),
            pl.BlockSpec((2, h.shape[1]), lambda i: (0, 0)),
        ],
        out_specs=[
            pl.BlockSpec((rb, 64), lambda i: (i, 0)),
            pl.BlockSpec((rb, 2), lambda i: (i, 0)),
            pl.BlockSpec((rb, 2), lambda i: (i, 0)),
        ],
        out_shape=[
            jax.ShapeDtypeStruct((n, 64), jnp.float32),
            jax.ShapeDtypeStruct((n_pad, 2), jnp.float32),
            jax.ShapeDtypeStruct((n, 2), jnp.float32),
        ],
    )(h, stats, m, gb)


def _edge_body(src_hbm, dst_hbm, ew_hbm, xtab_hbm, adst_hbm, asrc_hbm, c2_hbm,
               acc_hbm,
               bs, bd, bw, psrc, pdl, pew, xrows, arows, srows, ppe0, ppe1,
               zrows, c2_v, acc_sp, adst_sp, sem1, sem2, sem3):
    core = lax.axis_index("c")
    tid = lax.axis_index("s")
    e_total = src_hbm.shape[0]
    per_tile = e_total // NSUB
    nblk = per_tile // EBLK
    vpb = EBLK // 16

    pltpu.sync_copy(c2_hbm, c2_v)
    c2v = c2_v[pl.ds(0, 16)]
    c0b = jnp.full((16,), c2v[0], jnp.float32)
    c1b = jnp.full((16,), c2v[1], jnp.float32)
    iota16 = lax.iota(jnp.int32, 16)
    zero16 = jnp.zeros((16,), jnp.float32)
    one16 = jnp.ones((16,), jnp.float32)
    zero16i = jnp.zeros((16,), jnp.int32)
    col60 = jnp.full((16,), 60, jnp.int32)
    col61 = jnp.full((16,), 61, jnp.int32)
    acol0 = jnp.zeros((16,), jnp.int32)
    acol1 = jnp.full((16,), 1, jnp.int32)
    bn16 = jnp.full((16,), B_N, jnp.int32)
    slope16 = jnp.full((16,), NEG_SLOPE, jnp.float32)
    pm16 = jnp.full((16,), PROWS - 1, jnp.int32)
    cm16 = jnp.full((16,), DRAIN - 1, jnp.int32)
    sh16 = jnp.full((16,), 8, jnp.int32)

    def zr_body(r, carry):
        for k in range(4):
            zrows[r, pl.ds(16 * k, 16)] = zero16
        return carry

    lax.fori_loop(0, 128, zr_body, 0)

    def drain(r):
        cp1 = pltpu.async_copy(xtab_hbm.at[psrc.at[r]], xrows, sem1)
        cp2 = pltpu.async_copy(adst_sp.at[pdl.at[r]], arows, sem2)
        cp3 = pltpu.async_copy(asrc_hbm.at[psrc.at[r]], srows, sem3)
        cp1.wait()
        cp2.wait()
        cp3.wait()
        def pe_body(j, carry):
            ei = jnp.full((16,), j * 16, jnp.int32) + iota16
            a0 = plsc.load_gather(srows, [ei, acol0])
            a1 = plsc.load_gather(srows, [ei, acol1])
            b0 = plsc.load_gather(arows, [ei, acol0])
            b1 = plsc.load_gather(arows, [ei, acol1])
            w = pew[r, pl.ds(j * 16, 16)]
            l0 = a0 + b0 + c0b * w
            l0 = jnp.where(l0 >= zero16, l0, slope16 * l0)
            l1 = a1 + b1 + c1b * w
            l1 = jnp.where(l1 >= zero16, l1, slope16 * l1)
            ppe0[pl.ds(j * 16, 16)] = jnp.exp(l0)
            ppe1[pl.ds(j * 16, 16)] = jnp.exp(l1)
            return carry

        lax.fori_loop(0, DRAIN // 16, pe_body, 0)

        def mul_body(j, carry):
            ev = jnp.full((16,), j * 16, jnp.int32) + iota16
            p0v = ppe0[pl.ds(j * 16, 16)]
            p1v = ppe1[pl.ds(j * 16, 16)]
            wv = pew[r, pl.ds(j * 16, 16)]
            i14 = jnp.full((16,), 14, jnp.int32)
            for e2 in range(16):
                e = j * 16 + e2
                p0 = jnp.full((16,), p0v[e2], jnp.float32)
                p1 = jnp.full((16,), p1v[e2], jnp.float32)
                c1 = jnp.where(iota16 < i14, p0, p1)
                v0 = xrows[e, pl.ds(0, 16)]
                xrows[e, pl.ds(0, 16)] = v0 * p0
                v1 = xrows[e, pl.ds(16, 16)]
                xrows[e, pl.ds(16, 16)] = v1 * c1
                v2 = xrows[e, pl.ds(32, 16)]
                xrows[e, pl.ds(32, 16)] = v2 * p1
                v3 = xrows[e, pl.ds(48, 16)]
                xrows[e, pl.ds(48, 16)] = v3 * p1
            plsc.store_scatter(xrows, [ev, col60], p0v)
            plsc.store_scatter(xrows, [ev, col61], p1v)
            plsc.store_scatter(xrows, [ev, jnp.full((16,), 62, jnp.int32)],
                               one16)
            plsc.store_scatter(xrows, [ev, jnp.full((16,), 63, jnp.int32)],
                               wv)
            return carry

        lax.fori_loop(0, DRAIN // 16, mul_body, 0)
        pltpu.sync_copy(xrows, acc_sp.at[pdl.at[r]], add=True)

    for p in range(NPASS):
        base = (NPASS * core + p) * B_N
        for q in range(SHARE // 128):
            pltpu.sync_copy(zrows, acc_sp.at[pl.ds(tid * SHARE + q * 128, 128)])
        if SHARE % 128:
            pltpu.sync_copy(
                zrows.at[pl.ds(0, SHARE % 128)],
                acc_sp.at[pl.ds(tid * SHARE + SHARE - SHARE % 128,
                                SHARE % 128)])
        pltpu.sync_copy(adst_hbm.at[pl.ds(base + tid * SHARE, SHARE)],
                        adst_sp.at[pl.ds(tid * SHARE, SHARE)])
        plsc.subcore_barrier()

        def blk_body(blk, carry):
            pend, done = carry
            off = tid * per_tile + blk * EBLK
            cpa = pltpu.async_copy(src_hbm.at[pl.ds(off, EBLK)], bs, sem1)
            cpb = pltpu.async_copy(dst_hbm.at[pl.ds(off, EBLK)], bd, sem2)
            cpc = pltpu.async_copy(ew_hbm.at[pl.ds(off, EBLK)], bw, sem3)
            cpa.wait()
            cpb.wait()
            cpc.wait()

            baseb = jnp.full((16,), base, jnp.int32)

            def vreg_body(i, pend):
                s = bs[pl.ds(i * 16, 16)]
                d = bd[pl.ds(i * 16, 16)]
                w = bw[pl.ds(i * 16, 16)]
                dl = d - baseb
                m = (dl >= zero16i) & (dl < bn16)
                mi = m.astype(jnp.int32)
                cum = plsc.cumsum(mi)
                pos = jnp.full((16,), pend, jnp.int32) + cum - mi
                row = (pos >> sh16) & pm16
                col = pos & cm16
                plsc.store_scatter(psrc, [row, col], s, mask=m)
                plsc.store_scatter(pdl, [row, col], dl, mask=m)
                plsc.store_scatter(pew, [row, col], w, mask=m)
                return pend + cum[15]

            pend = lax.fori_loop(0, vpb, vreg_body, pend)

            def dcond(c):
                return c[0] - c[1] >= DRAIN

            def dbody(c):
                pend, done = c
                drain((done >> 8) & (PROWS - 1))
                return (pend, done + DRAIN)

            pend, done = lax.while_loop(dcond, dbody, (pend, done))
            return (pend, done)

        pend, done = lax.fori_loop(0, nblk, blk_body,
                                   (jnp.int32(0), jnp.int32(0)))

        @pl.when(pend > done)
        def _():
            for j in range(DRAIN // 16):
                pos = jnp.full((16,), pend + j * 16, jnp.int32) + iota16
                row = (pos >> sh16) & pm16
                col = pos & cm16
                plsc.store_scatter(psrc, [row, col], zero16i)
                plsc.store_scatter(pdl, [row, col], bn16)
                plsc.store_scatter(pew, [row, col], zero16)
            drain((done >> 8) & (PROWS - 1))

        plsc.subcore_barrier()
        for q in range(SHARE // 256):
            pltpu.sync_copy(acc_sp.at[pl.ds(tid * SHARE + q * 256, 256)],
                            acc_hbm.at[pl.ds(base + tid * SHARE + q * 256, 256)])
        if SHARE % 256:
            pltpu.sync_copy(
                acc_sp.at[pl.ds(tid * SHARE + SHARE - SHARE % 256,
                                SHARE % 256)],
                acc_hbm.at[pl.ds(base + tid * SHARE + SHARE - SHARE % 256,
                                 SHARE % 256)])
        if p + 1 < NPASS:
            plsc.subcore_barrier()


def _edge_phase(src, dst, ew, xtab, adst_pad, asrc_tab, c2pad):
    mesh = plsc.VectorSubcoreMesh(core_axis_name="c", subcore_axis_name="s")
    call = pl.kernel(
        _edge_body,
        out_type=jax.ShapeDtypeStruct((ACC_ROWS, 64), jnp.float32),
        mesh=mesh,
        compiler_params=pltpu.CompilerParams(
            needs_layout_passes=False, use_tc_tiling_on_sc=False),
        scratch_types=[
            pltpu.VMEM((EBLK,), jnp.int32),
            pltpu.VMEM((EBLK,), jnp.int32),
            pltpu.VMEM((EBLK,), jnp.float32),
            pltpu.VMEM((PROWS, DRAIN), jnp.int32),
            pltpu.VMEM((PROWS, DRAIN), jnp.int32),
            pltpu.VMEM((PROWS, DRAIN), jnp.float32),
            pltpu.VMEM((DRAIN, 64), jnp.float32),
            pltpu.VMEM((DRAIN, 2), jnp.float32),
            pltpu.VMEM((DRAIN, 2), jnp.float32),
            pltpu.VMEM((DRAIN,), jnp.float32),
            pltpu.VMEM((DRAIN,), jnp.float32),
            pltpu.VMEM((128, 64), jnp.float32),
            pltpu.VMEM((16,), jnp.float32),
            pltpu.VMEM_SHARED((B_N + 8, 64), jnp.float32),
            pltpu.VMEM_SHARED((B_N + 8, 2), jnp.float32),
            pltpu.SemaphoreType.DMA,
            pltpu.SemaphoreType.DMA,
            pltpu.SemaphoreType.DMA,
        ],
    )
    return call(src, dst, ew, xtab, adst_pad, asrc_tab, c2pad)


def _head_body(acc_ref, xt_ref, c2_ref, wb_ref, a_ref, b_ref):
    acc = acc_ref[...]
    xt = xt_ref[...]
    rb = acc.shape[0]
    c2 = c2_ref[0, :]

    counts = acc[:, 62]
    wsum = acc[:, 63]
    la = wsum / jnp.maximum(counts, 1.0)
    asrc = xt[:, 60:62]
    adst = xt[:, 62:64]
    pes = jnp.exp(_leaky(asrc + adst + la[:, None] * c2[None, 0:2]))
    den = acc[:, 60:62] + pes
    pe_cols = jnp.concatenate(
        [jnp.broadcast_to(pes[:, 0:1], (rb, C)),
         jnp.broadcast_to(pes[:, 1:2], (rb, C))], axis=1)
    den_cols = jnp.concatenate(
        [jnp.broadcast_to(den[:, 0:1], (rb, C)),
         jnp.broadcast_to(den[:, 1:2], (rb, C))], axis=1)
    out = (acc[:, 0:HC] + pe_cols * xt[:, 0:HC]) / den_cols

    wb = wb_ref[...]
    hh = jnp.maximum(out + wb[0:1, 0:HC], 0.0)
    hh = jnp.maximum(
        jnp.dot(hh, wb[1:61, 0:10], preferred_element_type=jnp.float32)
        + wb[61:62, 0:10], 0.0)
    hh = jnp.maximum(
        jnp.dot(hh, wb[62:72, 0:10], preferred_element_type=jnp.float32)
        + wb[72:73, 0:10], 0.0)
    hh = (jnp.dot(hh, wb[73:83, 0:10], preferred_element_type=jnp.float32)
          + wb[83:84, 0:10])
    ab = (jnp.dot(hh, wb[84:94, 0:2], preferred_element_type=jnp.float32)
          + wb[94:95, 0:2])
    ab = jnp.where(ab > 0, ab, jnp.exp(jnp.minimum(ab, 0.0)) - 1.0) + 1.0
    a_ref[...] = ab[:, 0:1]
    b_ref[...] = ab[:, 1:2]


def _head(acc, xtab, c2, wb, rb):
    n = xtab.shape[0]
    grid = (n + rb - 1) // rb
    return pl.pallas_call(
        _head_body,
        grid=(grid,),
        in_specs=[
            pl.BlockSpec((rb, 64), lambda i: (i, 0)),
            pl.BlockSpec((rb, 64), lambda i: (i, 0)),
            pl.BlockSpec((1, 2), lambda i: (0, 0)),
            pl.BlockSpec((95, 64), lambda i: (0, 0)),
        ],
        out_specs=[
            pl.BlockSpec((rb, 1), lambda i: (i, 0)),
            pl.BlockSpec((rb, 1), lambda i: (i, 0)),
        ],
        out_shape=[
            jax.ShapeDtypeStruct((n, 1), jnp.float32),
            jax.ShapeDtypeStruct((n, 1), jnp.float32),
        ],
    )(acc, xtab, c2, wb)


def _pack_head_weights(params):
    rows = []

    def pad(row2d):
        r, c = row2d.shape
        return jnp.pad(row2d, ((0, 0), (0, 64 - c)))

    rows.append(pad(params['conv_bias'][None, :]))
    rows.append(pad(params['fc1_W'].T))
    rows.append(pad(params['fc1_b'][None, :]))
    rows.append(pad(params['fc2_W'].T))
    rows.append(pad(params['fc2_b'][None, :]))
    rows.append(pad(params['fc3_W'].T))
    rows.append(pad(params['fc3_b'][None, :]))
    fc45 = jnp.concatenate([params['fc4_W'], params['fc5_W']], axis=0)
    rows.append(pad(fc45.T))
    fc45b = jnp.concatenate([params['fc4_b'], params['fc5_b']])[None, :]
    rows.append(pad(fc45b))
    return jnp.concatenate(rows, axis=0)


def kernel(h, edge_index, edge_weight, params):
    n, in_dim = h.shape

    wt = params['lin_W'].T
    att_s = params['att_src'][0]
    att_d = params['att_dst'][0]
    sel_s = jnp.zeros((HC, H), jnp.float32)
    sel_d = jnp.zeros((HC, H), jnp.float32)
    for hh in range(H):
        sel_s = sel_s.at[hh * C:(hh + 1) * C, hh].set(att_s[hh])
        sel_d = sel_d.at[hh * C:(hh + 1) * C, hh].set(att_d[hh])
    m = jnp.concatenate([wt, wt @ sel_s, wt @ sel_d], axis=1)
    gb = jnp.stack([params['bn_gamma'], params['bn_beta']])
    le = params['lin_edge_W'][:, 0].reshape(H, C)
    c2 = jnp.sum(le * params['att_edge'][0], axis=-1)[None, :]

    stats = _stats(h)
    xtab, adst_pad, asrc_tab = _xtab(h, stats, m, gb, rb=8192,
                                     n_pad=ACC_ROWS)

    src, dst = edge_index[0], edge_index[1]
    ew = edge_weight[:, 0]
    c2pad = jnp.pad(c2[0], (0, 14))
    acc_full = _edge_phase(src, dst, ew, xtab, adst_pad, asrc_tab, c2pad)

    wb = _pack_head_weights(params)
    a_out, b_out = _head(acc_full, xtab, c2, wb, rb=8192)
    return a_out, b_out
